# baseline (speedup 1.0000x reference)
import functools

import jax
import jax.numpy as jnp
from jax.experimental import pallas as pl
from jax.experimental.pallas import tpu as pltpu

_BRA = 400
_BRB = 1000
_FC = 128


def _xw_body(nrows, xt_ref, w_ref, o_ref, acc_ref):
    j = pl.program_id(0)
    rowid = j * _FC + jax.lax.broadcasted_iota(jnp.int32, (_FC, 1), 0)
    xt = jnp.where(rowid < nrows, xt_ref[...], jnp.bfloat16(0.0))
    part = jax.lax.dot_general(
        xt, w_ref[...].astype(jnp.bfloat16),
        (((0,), (0,)), ((), ())), preferred_element_type=jnp.float32)

    @pl.when(j == 0)
    def _():
        acc_ref[...] = part

    @pl.when(j > 0)
    def _():
        acc_ref[...] += part

    @pl.when(j == pl.num_programs(0) - 1)
    def _():
        o_ref[...] = acc_ref[...].astype(jnp.bfloat16)


def _l1_body(adj_ref, s_ref, b_ref, w2_ref, w3_ref, o_ref, a16_ref):
    a16 = adj_ref[...].astype(jnp.bfloat16)
    a16_ref[...] = a16
    h = jnp.dot(a16, s_ref[...], preferred_element_type=jnp.float32)
    h = jnp.maximum(h + b_ref[...], 0.0)
    hw2 = jnp.dot(h, w2_ref[...], preferred_element_type=jnp.float32)
    o_ref[...] = jnp.dot(hw2, w3_ref[...],
                         preferred_element_type=jnp.float32).astype(jnp.bfloat16)


def _cd_body(a16_ref, u_ref, b2_ref, w3_ref, b3_ref, o_ref, t_ref):
    i = pl.program_id(0)
    nh = pl.num_programs(0) // 2
    step = jax.lax.rem(i, nh)

    @pl.when(i < nh)
    def _():
        c = jnp.dot(b2_ref[...], w3_ref[...],
                    preferred_element_type=jnp.float32)
        h = jnp.dot(a16_ref[...], u_ref[...],
                    preferred_element_type=jnp.float32)
        t_ref[pl.ds(step * _BRB, _BRB), :] = (h + c).astype(jnp.bfloat16)

    @pl.when(i >= nh)
    def _():
        h = jnp.dot(a16_ref[...], t_ref[...],
                    preferred_element_type=jnp.float32) + b3_ref[...]
        m = jnp.max(h, axis=1, keepdims=True)
        lse = jnp.log(jnp.sum(jnp.exp(h - m), axis=1, keepdims=True))
        o_ref[...] = h - m - lse


def kernel(x, adj, W1, b1, W2, b2, W3, b3):
    N, F = x.shape
    d1 = W1.shape[1]
    d2 = W2.shape[1]
    d3 = W3.shape[1]
    b1r = b1.reshape(1, d1)
    b2r = b2.reshape(1, d2)
    b3r = b3.reshape(1, d3)

    row = lambda i: (i, 0)
    const = lambda i: (0, 0)

    FP = ((F + _FC - 1) // _FC) * _FC
    xtp = jnp.swapaxes(x, 0, 1).astype(jnp.bfloat16)
    w1p = jnp.pad(W1, ((0, FP - F), (0, 0)))
    s1 = pl.pallas_call(
        functools.partial(_xw_body, F),
        grid=(FP // _FC,),
        in_specs=[pl.BlockSpec((_FC, N), row),
                  pl.BlockSpec((_FC, d1), row)],
        out_specs=pl.BlockSpec((N, d1), const),
        out_shape=jax.ShapeDtypeStruct((N, d1), jnp.bfloat16),
        scratch_shapes=[pltpu.VMEM((N, d1), jnp.float32)],
    )(xtp, w1p)

    u, adj16 = pl.pallas_call(
        _l1_body,
        grid=(N // _BRA,),
        in_specs=[pl.BlockSpec((_BRA, N), row),
                  pl.BlockSpec((N, d1), const),
                  pl.BlockSpec((1, d1), const),
                  pl.BlockSpec((d1, d2), const),
                  pl.BlockSpec((d2, d3), const)],
        out_specs=[pl.BlockSpec((_BRA, d3), row),
                   pl.BlockSpec((_BRA, N), row)],
        out_shape=[jax.ShapeDtypeStruct((N, d3), jnp.bfloat16),
                   jax.ShapeDtypeStruct((N, N), jnp.bfloat16)],
    )(adj, s1, b1r, W2, W3)

    nh = N // _BRB
    return pl.pallas_call(
        _cd_body,
        grid=(2 * nh,),
        in_specs=[pl.BlockSpec((_BRB, N),
                               lambda i: (jax.lax.rem(i, nh), 0)),
                  pl.BlockSpec((N, d3), const),
                  pl.BlockSpec((1, d2), const),
                  pl.BlockSpec((d2, d3), const),
                  pl.BlockSpec((1, d3), const)],
        out_specs=pl.BlockSpec((_BRB, d3),
                               lambda i: (jnp.maximum(i - nh, 0), 0)),
        out_shape=jax.ShapeDtypeStruct((N, d3), jnp.float32),
        scratch_shapes=[pltpu.VMEM((N, d3), jnp.bfloat16)],
    )(adj16, u, b2r, W3, b3r)


# speedup vs baseline: 1.0685x; 1.0685x over previous
import functools

import jax
import jax.numpy as jnp
from jax.experimental import pallas as pl
from jax.experimental.pallas import tpu as pltpu

_BRA = 400
_BRB = 1000
_FC = 128


def _xw_body(nrows, xt_ref, w_ref, o_ref, acc_ref):
    j = pl.program_id(0)
    rowid = j * _FC + jax.lax.broadcasted_iota(jnp.int32, (_FC, 1), 0)
    xt = jnp.where(rowid < nrows, xt_ref[...], 0.0)
    part = jax.lax.dot_general(
        xt.astype(jnp.bfloat16), w_ref[...].astype(jnp.bfloat16),
        (((0,), (0,)), ((), ())), preferred_element_type=jnp.float32)

    @pl.when(j == 0)
    def _():
        acc_ref[...] = part

    @pl.when(j > 0)
    def _():
        acc_ref[...] += part

    @pl.when(j == pl.num_programs(0) - 1)
    def _():
        o_ref[...] = acc_ref[...].astype(jnp.bfloat16)


def _l1_body(adj_ref, s_ref, b_ref, w2_ref, w3_ref, o_ref, a16_ref):
    a16 = adj_ref[...].astype(jnp.bfloat16)
    a16_ref[...] = a16
    h = jnp.dot(a16, s_ref[...], preferred_element_type=jnp.float32)
    h = jnp.maximum(h + b_ref[...], 0.0)
    hw2 = jnp.dot(h, w2_ref[...], preferred_element_type=jnp.float32)
    o_ref[...] = jnp.dot(hw2, w3_ref[...],
                         preferred_element_type=jnp.float32).astype(jnp.bfloat16)


def _cd_body(a16_ref, u_ref, b2_ref, w3_ref, b3_ref, o_ref, t_ref):
    i = pl.program_id(0)
    nh = pl.num_programs(0) // 2
    step = jax.lax.rem(i, nh)

    @pl.when(i < nh)
    def _():
        c = jnp.dot(b2_ref[...], w3_ref[...],
                    preferred_element_type=jnp.float32)
        h = jnp.dot(a16_ref[...], u_ref[...],
                    preferred_element_type=jnp.float32)
        t_ref[pl.ds(step * _BRB, _BRB), :] = (h + c).astype(jnp.bfloat16)

    @pl.when(i >= nh)
    def _():
        h = jnp.dot(a16_ref[...], t_ref[...],
                    preferred_element_type=jnp.float32) + b3_ref[...]
        m = jnp.max(h, axis=1, keepdims=True)
        lse = jnp.log(jnp.sum(jnp.exp(h - m), axis=1, keepdims=True))
        o_ref[...] = h - m - lse


def kernel(x, adj, W1, b1, W2, b2, W3, b3):
    N, F = x.shape
    d1 = W1.shape[1]
    d2 = W2.shape[1]
    d3 = W3.shape[1]
    b1r = b1.reshape(1, d1)
    b2r = b2.reshape(1, d2)
    b3r = b3.reshape(1, d3)

    row = lambda i: (i, 0)
    const = lambda i: (0, 0)

    FP = ((F + _FC - 1) // _FC) * _FC
    xtp = jnp.swapaxes(x, 0, 1)
    w1p = jnp.pad(W1, ((0, FP - F), (0, 0)))
    s1 = pl.pallas_call(
        functools.partial(_xw_body, F),
        grid=(FP // _FC,),
        in_specs=[pl.BlockSpec((_FC, N), row),
                  pl.BlockSpec((_FC, d1), row)],
        out_specs=pl.BlockSpec((N, d1), const),
        out_shape=jax.ShapeDtypeStruct((N, d1), jnp.bfloat16),
        scratch_shapes=[pltpu.VMEM((N, d1), jnp.float32)],
    )(xtp, w1p)

    u, adj16 = pl.pallas_call(
        _l1_body,
        grid=(N // _BRA,),
        in_specs=[pl.BlockSpec((_BRA, N), row),
                  pl.BlockSpec((N, d1), const),
                  pl.BlockSpec((1, d1), const),
                  pl.BlockSpec((d1, d2), const),
                  pl.BlockSpec((d2, d3), const)],
        out_specs=[pl.BlockSpec((_BRA, d3), row),
                   pl.BlockSpec((_BRA, N), row)],
        out_shape=[jax.ShapeDtypeStruct((N, d3), jnp.bfloat16),
                   jax.ShapeDtypeStruct((N, N), jnp.bfloat16)],
    )(adj, s1, b1r, W2, W3)

    nh = N // _BRB
    return pl.pallas_call(
        _cd_body,
        grid=(2 * nh,),
        in_specs=[pl.BlockSpec((_BRB, N),
                               lambda i: (jax.lax.rem(i, nh), 0)),
                  pl.BlockSpec((N, d3), const),
                  pl.BlockSpec((1, d2), const),
                  pl.BlockSpec((d2, d3), const),
                  pl.BlockSpec((1, d3), const)],
        out_specs=pl.BlockSpec((_BRB, d3),
                               lambda i: (jnp.maximum(i - nh, 0), 0)),
        out_shape=jax.ShapeDtypeStruct((N, d3), jnp.float32),
        scratch_shapes=[pltpu.VMEM((N, d3), jnp.bfloat16)],
    )(adj16, u, b2r, W3, b3r)
